# baseline (speedup 1.0000x reference)
import jax
import jax.numpy as jnp
from jax import lax
from jax.experimental import pallas as pl
from jax.experimental.pallas import tpu as pltpu
from jax.experimental.pallas import tpu_sc as plsc

B = 16384
F = 26
VOCAB = 1000000
NC, NS = 2, 16
NW = NC * NS
BW = B // NW
NIDX = BW * F
NROW = NIDX // 128
JB = BW // 128


def _body(xt_hbm, tab_hbm, out_hbm, idx_v, vals_v, out_v, sem):
    wid = lax.axis_index("s") * NC + lax.axis_index("c")
    pltpu.sync_copy(xt_hbm.at[wid], idx_v)

    def add_off(r, _):
        off = (r // JB) * VOCAB
        for l in range(8):
            sl = pl.ds(l * 16, 16)
            idx_v[r, sl] = idx_v[r, sl] + off
        return 0

    lax.fori_loop(0, NROW, add_off, 0)

    def gat(g, _):
        for i in range(8):
            r = g * 8 + i
            pltpu.make_async_copy(
                tab_hbm.at[idx_v.at[r]],
                vals_v.at[pl.ds(r * 128, 128)],
                sem,
            ).start()
        for i in range(8):
            r = g * 8 + i
            pltpu.make_async_copy(
                tab_hbm.at[idx_v.at[r]],
                vals_v.at[pl.ds(r * 128, 128)],
                sem,
            ).wait()
        return 0

    lax.fori_loop(0, NROW // 8, gat, 0)

    def red(l, _):
        j0 = l * 16
        acc = vals_v[pl.ds(j0, 16)]
        for f in range(1, F):
            acc = acc + vals_v[pl.ds(f * BW + j0, 16)]
        out_v[pl.ds(j0, 16)] = 1.0 / (1.0 + jnp.exp(-acc))
        return 0

    lax.fori_loop(0, BW // 16, red, 0)

    pltpu.sync_copy(out_v, out_hbm.at[pl.ds(wid * BW, BW)])


@jax.jit
def kernel(X, linear_tables):
    xt = X.reshape(NW, BW, F).transpose(0, 2, 1).reshape(NW, NROW, 128)
    tab = linear_tables.reshape(F * VOCAB)
    mesh = plsc.VectorSubcoreMesh(
        core_axis_name="c", subcore_axis_name="s",
        num_cores=NC, num_subcores=NS)
    run = pl.kernel(
        _body,
        out_type=jax.ShapeDtypeStruct((B,), jnp.float32),
        mesh=mesh,
        scratch_types=[
            pltpu.VMEM((NROW, 128), jnp.int32),
            pltpu.VMEM((NIDX,), jnp.float32),
            pltpu.VMEM((BW,), jnp.float32),
            pltpu.SemaphoreType.DMA,
        ],
    )
    return run(xt, tab).reshape(B, 1)
